# baseline (speedup 1.0000x reference)
import math

import jax
import jax.numpy as jnp
import numpy as np
from jax.experimental import pallas as pl
from jax.experimental.pallas import tpu as pltpu

_D_MODEL = 768
_GLOVE_DIM = 300
_SEQ = 200
_TI = 8
_VT = 10000
_VOCAB = 400000
_V0 = 240000
_NS = (_VOCAB - _V0) // _VT
_ROW_BYTES = _GLOVE_DIM * 4


def _pe_const():
    position = np.arange(0, _SEQ, dtype=np.float32)[:, None]
    div_term = np.exp(
        np.arange(0, _D_MODEL, 2, dtype=np.float32)
        * (-math.log(10000.0) / _D_MODEL)
    )
    pe = np.zeros((_SEQ, _D_MODEL), dtype=np.float32)
    pe[:, 0::2] = np.sin(position * div_term)
    pe[:, 1::2] = np.cos(position * div_term)
    return pe


_PE = _pe_const()


def _fused_body(
    toks_ref, glove_hbm, stream_ref, w_ref, b_ref, pe_ref, out_ref,
    vec_ref, y_ref, sem,
):
    k = pl.program_id(0)

    @pl.when(k == 0)
    def _():
        for j in range(_SEQ):
            tj = toks_ref[0, j]

            @pl.when(tj < _V0)
            def _():
                pltpu.make_async_copy(
                    glove_hbm.at[pl.ds(tj, 1)], vec_ref.at[pl.ds(j, 1)], sem
                ).start()

    @pl.when(k < _NS)
    def _():
        base = _V0 + k * _VT
        for j in range(_SEQ):
            lt = toks_ref[0, j] - base

            @pl.when(jnp.logical_and(lt >= 0, lt < _VT))
            def _():
                vec_ref[pl.ds(j, 1), :] = stream_ref[pl.ds(lt, 1), :]

    @pl.when(k == _NS)
    def _():
        for j in range(_SEQ):
            tj = toks_ref[0, j]

            @pl.when(tj < _V0)
            def _():
                pltpu.make_async_copy(
                    glove_hbm.at[pl.ds(tj, 1)], vec_ref.at[pl.ds(j, 1)], sem
                ).wait()

        y_ref[...] = (
            jnp.dot(vec_ref[...], w_ref[...], preferred_element_type=jnp.float32)
            + b_ref[...]
        )

    @pl.when(k >= _NS)
    def _():
        out_ref[...] = y_ref[...][None, :, :] + pe_ref[...][:, None, :]


@jax.jit
def kernel(tokens, glove_table, W, b):
    S = _SEQ

    pe = jnp.asarray(_PE)
    b2 = b.reshape(1, _D_MODEL)
    toks2 = tokens.reshape(1, S)

    out = pl.pallas_call(
        _fused_body,
        grid=(_NS + S // _TI,),
        in_specs=[
            pl.BlockSpec(memory_space=pltpu.SMEM),
            pl.BlockSpec(memory_space=pltpu.HBM),
            pl.BlockSpec(
                (_VT, _GLOVE_DIM),
                lambda k: (_V0 // _VT + jnp.minimum(k, _NS - 1), 0),
            ),
            pl.BlockSpec((_GLOVE_DIM, _D_MODEL), lambda k: (0, 0)),
            pl.BlockSpec((1, _D_MODEL), lambda k: (0, 0)),
            pl.BlockSpec(
                (_TI, _D_MODEL), lambda k: (jnp.maximum(k - _NS, 0), 0)
            ),
        ],
        out_specs=pl.BlockSpec(
            (_TI, S, _D_MODEL), lambda k: (jnp.maximum(k - _NS, 0), 0, 0)
        ),
        out_shape=jax.ShapeDtypeStruct((S, S, _D_MODEL), jnp.float32),
        scratch_shapes=[
            pltpu.VMEM((S, _GLOVE_DIM), jnp.float32),
            pltpu.VMEM((S, _D_MODEL), jnp.float32),
            pltpu.SemaphoreType.DMA,
        ],
    )(toks2, glove_table, glove_table, W, b2, pe)

    return out


# speedup vs baseline: 1.1361x; 1.1361x over previous
import math

import jax
import jax.numpy as jnp
import numpy as np
from jax.experimental import pallas as pl
from jax.experimental.pallas import tpu as pltpu

_D_MODEL = 768
_GLOVE_DIM = 300
_SEQ = 200
_TI = 8


def _pe_const():
    position = np.arange(0, _SEQ, dtype=np.float32)[:, None]
    div_term = np.exp(
        np.arange(0, _D_MODEL, 2, dtype=np.float32)
        * (-math.log(10000.0) / _D_MODEL)
    )
    pe = np.zeros((_SEQ, _D_MODEL), dtype=np.float32)
    pe[:, 0::2] = np.sin(position * div_term)
    pe[:, 1::2] = np.cos(position * div_term)
    return pe


_PE = _pe_const()


def _fused_body(toks_ref, glove_hbm, w_ref, b_ref, pe_ref, out_ref, vec_ref, y_ref, sem):
    i = pl.program_id(0)

    @pl.when(i == 0)
    def _():
        copies = [
            pltpu.make_async_copy(
                glove_hbm.at[pl.ds(toks_ref[0, j], 1)],
                vec_ref.at[pl.ds(j, 1)],
                sem,
            )
            for j in range(_SEQ)
        ]
        for c in copies:
            c.start()
        for c in copies:
            c.wait()
        y_ref[...] = (
            jnp.dot(vec_ref[...], w_ref[...], preferred_element_type=jnp.float32)
            + b_ref[...]
        )

    out_ref[...] = y_ref[...][None, :, :] + pe_ref[...][:, None, :]


@jax.jit
def kernel(tokens, glove_table, W, b):
    S = _SEQ

    pe = jnp.asarray(_PE)
    b2 = b.reshape(1, _D_MODEL)
    toks2 = tokens.reshape(1, S)

    out = pl.pallas_call(
        _fused_body,
        grid=(S // _TI,),
        in_specs=[
            pl.BlockSpec(memory_space=pltpu.SMEM),
            pl.BlockSpec(memory_space=pltpu.HBM),
            pl.BlockSpec((_GLOVE_DIM, _D_MODEL), lambda i: (0, 0)),
            pl.BlockSpec((1, _D_MODEL), lambda i: (0, 0)),
            pl.BlockSpec((_TI, _D_MODEL), lambda i: (i, 0)),
        ],
        out_specs=pl.BlockSpec((_TI, S, _D_MODEL), lambda i: (i, 0, 0)),
        out_shape=jax.ShapeDtypeStruct((S, S, _D_MODEL), jnp.float32),
        scratch_shapes=[
            pltpu.VMEM((S, _GLOVE_DIM), jnp.float32),
            pltpu.VMEM((S, _D_MODEL), jnp.float32),
            pltpu.SemaphoreType.DMA,
        ],
    )(toks2, glove_table, W, b2, pe)

    return out
